# Initial kernel scaffold; baseline (speedup 1.0000x reference)
import jax
import jax.numpy as jnp
from jax.experimental import pallas as pl


def kernel(x):
    raise NotImplementedError("write your pallas kernel here")


# speedup vs baseline: 13.9636x; 13.9636x over previous
import functools

import jax
import jax.numpy as jnp
from jax.experimental import pallas as pl
from jax.experimental.pallas import tpu as pltpu

_K = 15
_N = 4096
_C = 64
_R = 512


def _knn_block_kernel(xb_ref, xa_ref, out_ref, d_ref):
    xb = xb_ref[0]
    xa = xa_ref[0]

    sq_all = jnp.sum(xa * xa, axis=0)
    sq_rows = jnp.sum(xb * xb, axis=0)

    g = jax.lax.dot_general(
        xb, xa, (((0,), (0,)), ((), ())),
        preferred_element_type=jnp.float32,
        precision=jax.lax.Precision.DEFAULT,
    )

    d = (sq_rows[:, None] + sq_all[None, :]) - 2.0 * g

    r = pl.program_id(1)
    base = r * _R
    col = jax.lax.broadcasted_iota(jnp.int32, (_R, _N), 1)
    row_g = jax.lax.broadcasted_iota(jnp.int32, (_R, _N), 0) + base
    d_ref[...] = jnp.where(col == row_g, jnp.inf, d)

    out_ref[1, 0, :, :] = (
        jax.lax.broadcasted_iota(jnp.int32, (_R, _K), 0) + base
    )

    kcol = jax.lax.broadcasted_iota(jnp.int32, (_R, _K), 1)

    def extract(k, acc):
        dk = d_ref[...]
        m = jnp.min(dk, axis=1)
        eq = dk == m[:, None]
        idx = jnp.min(jnp.where(eq, col, _N), axis=1)
        acc = jnp.where(kcol == k, idx[:, None], acc)
        d_ref[...] = jnp.where(col == idx[:, None], jnp.inf, dk)
        return acc

    acc0 = jnp.zeros((_R, _K), jnp.int32)
    out_ref[0, 0, :, :] = jax.lax.fori_loop(0, _K, extract, acc0)


@jax.jit
def kernel(x):
    x = jnp.squeeze(x, -1)
    B = x.shape[0]
    grid = (B, _N // _R)
    out = pl.pallas_call(
        _knn_block_kernel,
        grid=grid,
        in_specs=[
            pl.BlockSpec((1, _C, _R), lambda b, r: (b, 0, r)),
            pl.BlockSpec((1, _C, _N), lambda b, r: (b, 0, 0)),
        ],
        out_specs=pl.BlockSpec((2, 1, _R, _K), lambda b, r: (0, b, r, 0)),
        out_shape=jax.ShapeDtypeStruct((2, B, _N, _K), jnp.int32),
        scratch_shapes=[pltpu.VMEM((_R, _N), jnp.float32)],
    )(x, x)
    return out
